# Initial kernel scaffold; baseline (speedup 1.0000x reference)
import jax
import jax.numpy as jnp
from jax.experimental import pallas as pl


def kernel(x, positions, cw1, cb1, cw2, cb2, gnorm_w, mw1, mw2, mw3, gproj_w, pos_table, temperature, expert_load, expert_utilization, expert_priority):
    raise NotImplementedError("write your pallas kernel here")


# speedup vs baseline: 1.1265x; 1.1265x over previous
import math
import functools

import jax
import jax.numpy as jnp
from jax.experimental import pallas as pl

HI = jax.lax.Precision.HIGHEST


def _ctx_kernel(x_ref, cw1_ref, cb1_ref, cw2_ref, cb2_ref, pt_ref, pos_ref,
                feat_ref, nc_ref, *, tr, n_rows):
    i = pl.program_id(0)
    xb = x_ref[...]
    h = jnp.dot(xb, cw1_ref[...]) + cb1_ref[...]
    h = 0.5 * h * (1.0 + jax.lax.erf(h * (2.0 ** -0.5)))
    ctx = jnp.dot(h, cw2_ref[...]) + cb2_ref[...]
    pos = pos_ref[0, pl.ds(i * tr, tr)].reshape(tr, 1)
    vocab = pt_ref.shape[0]
    oh = (pos == jax.lax.broadcasted_iota(jnp.int32, (tr, vocab), 1))
    pe = jnp.dot(oh.astype(jnp.float32), pt_ref[...], precision=HI)
    feat = jnp.concatenate([ctx, pe], axis=1)
    feat_ref[...] = feat

    @pl.when(i == 0)
    def _():
        nc_ref[...] = jnp.zeros_like(nc_ref)

    nc_ref[...] += jnp.sum(feat, axis=0, keepdims=True) * (1.0 / n_rows)


def _gate_kernel(x_ref, feat_ref, gnorm_ref, mw1_ref, mw3_ref, mw2_ref,
                 gproj_ref, load_ref, temp_ref, prio_ref,
                 ts_ref, ti_ref, ssum_ref):
    i = pl.program_id(0)
    gcat = jnp.concatenate([x_ref[...], feat_ref[...]], axis=1)
    ms = jnp.mean(gcat * gcat, axis=1, keepdims=True)
    gi = gcat * jax.lax.rsqrt(ms + 1e-6) * gnorm_ref[...]
    h1 = jnp.dot(gi, mw1_ref[...])
    h3 = jnp.dot(gi, mw3_ref[...])
    m = jnp.dot(jax.nn.silu(h1) * h3, mw2_ref[...])
    gh = m + gi
    logits = jnp.dot(gh, gproj_ref[...]) + jnp.log(prio_ref[...])
    load = load_ref[...]
    lmean = jnp.mean(load)
    lstd = jnp.sqrt(jnp.mean((load - lmean) ** 2))
    imb = lstd / (lmean + 1e-6)
    temp = jnp.maximum(temp_ref[0, 0] * (1.0 + imb), 0.3)
    lt = logits / temp
    lmax = jnp.max(lt, axis=1, keepdims=True)
    ex = jnp.exp(lt - lmax)
    sc = ex / jnp.sum(ex, axis=1, keepdims=True)
    e_iota = jax.lax.broadcasted_iota(jnp.int32, sc.shape, 1)
    i1 = jnp.argmax(sc, axis=1).astype(jnp.int32)
    s1 = jnp.max(sc, axis=1)
    masked = jnp.where(e_iota == i1[:, None], -1.0, sc)
    i2 = jnp.argmax(masked, axis=1).astype(jnp.int32)
    s2 = jnp.max(masked, axis=1)
    ts_ref[...] = jnp.concatenate([s1[:, None], s2[:, None]], axis=1)
    ti_ref[...] = jnp.concatenate([i1[:, None], i2[:, None]], axis=1)

    @pl.when(i == 0)
    def _():
        ssum_ref[...] = jnp.zeros_like(ssum_ref)

    ssum_ref[...] += jnp.sum(sc, axis=0, keepdims=True)


def _dispatch_kernel(ts_ref, ti_ref, tsf_ref, tif_ref, prio_ref, util_ref,
                     ssum_ref, amask_ref, bpos_ref, ovf_ref, cnt_ref, aux_ref,
                     *, n_tok, k, n_exp, base_cap, chunk):
    nk = n_tok * k
    prio_c = jnp.clip(prio_ref[...], 1.0, 2.0)

    def prio_lookup(ev):
        out = jnp.zeros(ev.shape, jnp.float32)
        for v in range(n_exp):
            out = jnp.where(ev == v, prio_c[0, v], out)
        return out

    ef = tif_ref[...]
    wf = tsf_ref[...] * prio_lookup(ef)
    jj = jax.lax.broadcasted_iota(jnp.int32, (1, nk), 1)
    u = jnp.clip(util_ref[...], 0.1, 0.9)
    uf = jnp.mean(1.0 / (u + 0.1))
    capacity = jnp.clip((base_cap * uf).astype(jnp.int32), 4, 2048)
    rank_cols = []
    for kk in range(k):
        e_col = ti_ref[:, kk:kk + 1]
        w_col = ts_ref[:, kk:kk + 1] * prio_lookup(e_col)
        ii = jax.lax.broadcasted_iota(jnp.int32, (n_tok, 1), 0) * k + kk
        parts = []
        for c in range(n_tok // chunk):
            sl = slice(c * chunk, (c + 1) * chunk)
            wi = w_col[sl]
            ei = e_col[sl]
            iic = ii[sl]
            beats = (wf > wi) | ((wf == wi) & (jj < iic))
            cnt = jnp.sum(((ef == ei) & beats).astype(jnp.int32), axis=1,
                          keepdims=True)
            parts.append(cnt)
        rank_cols.append(jnp.concatenate(parts, axis=0))
    rank = jnp.concatenate(rank_cols, axis=1)
    assigned = rank < capacity
    amask_ref[...] = assigned.astype(jnp.int32)
    bpos_ref[...] = jnp.where(assigned, rank, 0).astype(jnp.int32)
    ovf_ref[...] = jnp.logical_not(
        jnp.any(assigned, axis=1, keepdims=True)).astype(jnp.int32)
    e_iota = jax.lax.broadcasted_iota(jnp.int32, (1, n_exp), 1)
    n_e = jnp.zeros((1, n_exp), jnp.int32)
    for kk in range(k):
        n_e += jnp.sum((ti_ref[:, kk:kk + 1] == e_iota).astype(jnp.int32),
                       axis=0, keepdims=True)
    ecnt = jnp.minimum(n_e, capacity)
    cnt_ref[...] = ecnt
    me = ssum_ref[...] * (1.0 / n_tok)
    ce = ecnt.astype(jnp.float32) / float(nk)
    aux_ref[...] = 0.01 * n_exp * jnp.sum(me * ce, keepdims=True).reshape(1, 1)


def kernel(x, positions, cw1, cb1, cw2, cb2, gnorm_w, mw1, mw2, mw3, gproj_w,
           pos_table, temperature, expert_load, expert_utilization,
           expert_priority):
    B, S, H = x.shape
    N = B * S
    E = gproj_w.shape[1]
    K = 2
    CTXF = cw2.shape[1] + pos_table.shape[1]
    GATE_DIM = gproj_w.shape[0]
    HID = mw1.shape[1]
    f32 = jnp.float32

    x_flat = x.reshape(N, H)
    pos_row = positions.reshape(1, N).astype(jnp.int32)

    TRA = 256
    feat, nc = pl.pallas_call(
        functools.partial(_ctx_kernel, tr=TRA, n_rows=N),
        grid=(N // TRA,),
        in_specs=[
            pl.BlockSpec((TRA, H), lambda i: (i, 0)),
            pl.BlockSpec(cw1.shape, lambda i: (0, 0)),
            pl.BlockSpec((1, cb1.shape[0]), lambda i: (0, 0)),
            pl.BlockSpec(cw2.shape, lambda i: (0, 0)),
            pl.BlockSpec((1, cb2.shape[0]), lambda i: (0, 0)),
            pl.BlockSpec(pos_table.shape, lambda i: (0, 0)),
            pl.BlockSpec((1, N), lambda i: (0, 0)),
        ],
        out_specs=[
            pl.BlockSpec((TRA, CTXF), lambda i: (i, 0)),
            pl.BlockSpec((1, CTXF), lambda i: (0, 0)),
        ],
        out_shape=[
            jax.ShapeDtypeStruct((N, CTXF), f32),
            jax.ShapeDtypeStruct((1, CTXF), f32),
        ],
    )(x_flat, cw1, cb1.reshape(1, -1), cw2, cb2.reshape(1, -1), pos_table,
      pos_row)

    TRB = 128
    ts, ti, ssum = pl.pallas_call(
        _gate_kernel,
        grid=(N // TRB,),
        in_specs=[
            pl.BlockSpec((TRB, H), lambda i: (i, 0)),
            pl.BlockSpec((TRB, CTXF), lambda i: (i, 0)),
            pl.BlockSpec((1, GATE_DIM), lambda i: (0, 0)),
            pl.BlockSpec(mw1.shape, lambda i: (0, 0)),
            pl.BlockSpec(mw3.shape, lambda i: (0, 0)),
            pl.BlockSpec(mw2.shape, lambda i: (0, 0)),
            pl.BlockSpec(gproj_w.shape, lambda i: (0, 0)),
            pl.BlockSpec((1, E), lambda i: (0, 0)),
            pl.BlockSpec((1, 1), lambda i: (0, 0)),
            pl.BlockSpec((1, E), lambda i: (0, 0)),
        ],
        out_specs=[
            pl.BlockSpec((TRB, K), lambda i: (i, 0)),
            pl.BlockSpec((TRB, K), lambda i: (i, 0)),
            pl.BlockSpec((1, E), lambda i: (0, 0)),
        ],
        out_shape=[
            jax.ShapeDtypeStruct((N, K), f32),
            jax.ShapeDtypeStruct((N, K), jnp.int32),
            jax.ShapeDtypeStruct((1, E), f32),
        ],
    )(x_flat, feat, gnorm_w.reshape(1, -1), mw1, mw3, mw2, gproj_w,
      expert_load.reshape(1, -1), temperature.reshape(1, 1),
      expert_priority.reshape(1, -1))

    base_cap = float(math.ceil(N * 1.25 / E))
    amask, bpos, ovf, ecnt, aux = pl.pallas_call(
        functools.partial(_dispatch_kernel, n_tok=N, k=K, n_exp=E,
                          base_cap=base_cap, chunk=512),
        grid=(1,),
        in_specs=[
            pl.BlockSpec((N, K), lambda i: (0, 0)),
            pl.BlockSpec((N, K), lambda i: (0, 0)),
            pl.BlockSpec((1, N * K), lambda i: (0, 0)),
            pl.BlockSpec((1, N * K), lambda i: (0, 0)),
            pl.BlockSpec((1, E), lambda i: (0, 0)),
            pl.BlockSpec((1, E), lambda i: (0, 0)),
            pl.BlockSpec((1, E), lambda i: (0, 0)),
        ],
        out_specs=[
            pl.BlockSpec((N, K), lambda i: (0, 0)),
            pl.BlockSpec((N, K), lambda i: (0, 0)),
            pl.BlockSpec((N, 1), lambda i: (0, 0)),
            pl.BlockSpec((1, E), lambda i: (0, 0)),
            pl.BlockSpec((1, 1), lambda i: (0, 0)),
        ],
        out_shape=[
            jax.ShapeDtypeStruct((N, K), jnp.int32),
            jax.ShapeDtypeStruct((N, K), jnp.int32),
            jax.ShapeDtypeStruct((N, 1), jnp.int32),
            jax.ShapeDtypeStruct((1, E), jnp.int32),
            jax.ShapeDtypeStruct((1, 1), f32),
        ],
    )(ts, ti, ts.reshape(1, N * K), ti.reshape(1, N * K),
      expert_priority.reshape(1, -1),
      expert_utilization.reshape(1, -1), ssum)

    return (ts, ti, amask.astype(bool), bpos, ovf.reshape(N).astype(bool),
            ecnt.reshape(E), aux.reshape(()), nc)


# speedup vs baseline: 1.1345x; 1.0071x over previous
import math
import functools

import jax
import jax.numpy as jnp
from jax.experimental import pallas as pl

bf16 = jnp.bfloat16
f32 = jnp.float32


def _ctx_kernel(x_ref, cw1_ref, cb1_ref, cw2_ref, cb2_ref, pth_ref, ptl_ref,
                pos_ref, feat_ref, nc_ref, *, tr, n_rows):
    i = pl.program_id(0)
    h = jnp.dot(x_ref[...], cw1_ref[...],
                preferred_element_type=f32) + cb1_ref[...]
    h = 0.5 * h * (1.0 + jax.lax.erf(h * (2.0 ** -0.5)))
    ctx = jnp.dot(h.astype(bf16), cw2_ref[...],
                  preferred_element_type=f32) + cb2_ref[...]
    pos = pos_ref[0, pl.ds(i * tr, tr)].reshape(tr, 1)
    vocab = pth_ref.shape[0]
    oh = (pos == jax.lax.broadcasted_iota(jnp.int32, (tr, vocab), 1))
    ohb = oh.astype(bf16)
    pe = (jnp.dot(ohb, pth_ref[...], preferred_element_type=f32) +
          jnp.dot(ohb, ptl_ref[...], preferred_element_type=f32))
    feat = jnp.concatenate([ctx, pe], axis=1)
    feat_ref[...] = feat

    @pl.when(i == 0)
    def _():
        nc_ref[...] = jnp.zeros_like(nc_ref)

    nc_ref[...] += jnp.sum(feat, axis=0, keepdims=True) * (1.0 / n_rows)


def _gate_kernel(x_ref, feat_ref, gnorm_ref, mw1_ref, mw3_ref, mw2_ref,
                 gproj_ref, load_ref, temp_ref, prio_ref,
                 ts_ref, ti_ref, ssum_ref):
    i = pl.program_id(0)
    gcat = jnp.concatenate([x_ref[...], feat_ref[...]], axis=1)
    ms = jnp.mean(gcat * gcat, axis=1, keepdims=True)
    gi = gcat * jax.lax.rsqrt(ms + 1e-6) * gnorm_ref[...]
    gib = gi.astype(bf16)
    h1 = jnp.dot(gib, mw1_ref[...], preferred_element_type=f32)
    h3 = jnp.dot(gib, mw3_ref[...], preferred_element_type=f32)
    hp = (jax.nn.silu(h1) * h3).astype(bf16)
    m = jnp.dot(hp, mw2_ref[...], preferred_element_type=f32)
    gh = m + gi
    logits = jnp.dot(gh.astype(bf16), gproj_ref[...],
                     preferred_element_type=f32) + jnp.log(prio_ref[...])
    load = load_ref[...]
    lmean = jnp.mean(load)
    lstd = jnp.sqrt(jnp.mean((load - lmean) ** 2))
    imb = lstd / (lmean + 1e-6)
    temp = jnp.maximum(temp_ref[0, 0] * (1.0 + imb), 0.3)
    lt = logits / temp
    lmax = jnp.max(lt, axis=1, keepdims=True)
    ex = jnp.exp(lt - lmax)
    sc = ex / jnp.sum(ex, axis=1, keepdims=True)
    e_iota = jax.lax.broadcasted_iota(jnp.int32, sc.shape, 1)
    i1 = jnp.argmax(sc, axis=1).astype(jnp.int32)
    s1 = jnp.max(sc, axis=1)
    masked = jnp.where(e_iota == i1[:, None], -1.0, sc)
    i2 = jnp.argmax(masked, axis=1).astype(jnp.int32)
    s2 = jnp.max(masked, axis=1)
    ts_ref[...] = jnp.concatenate([s1[:, None], s2[:, None]], axis=1)
    ti_ref[...] = jnp.concatenate([i1[:, None], i2[:, None]], axis=1)

    @pl.when(i == 0)
    def _():
        ssum_ref[...] = jnp.zeros_like(ssum_ref)

    ssum_ref[...] += jnp.sum(sc, axis=0, keepdims=True)


def _dispatch_kernel(ts_ref, ti_ref, tsf_ref, tif_ref, prio_ref, util_ref,
                     ssum_ref, amask_ref, bpos_ref, ovf_ref, cnt_ref, aux_ref,
                     *, n_tok, k, n_exp, base_cap, chunk):
    nk = n_tok * k
    prio_c = jnp.clip(prio_ref[...], 1.0, 2.0)

    def prio_lookup(ev):
        out = jnp.zeros(ev.shape, jnp.float32)
        for v in range(n_exp):
            out = jnp.where(ev == v, prio_c[0, v], out)
        return out

    ef = tif_ref[...]
    wf = tsf_ref[...] * prio_lookup(ef)
    jj = jax.lax.broadcasted_iota(jnp.int32, (1, nk), 1)
    u = jnp.clip(util_ref[...], 0.1, 0.9)
    uf = jnp.mean(1.0 / (u + 0.1))
    capacity = jnp.clip((base_cap * uf).astype(jnp.int32), 4, 2048)
    rank_cols = []
    for kk in range(k):
        e_col = ti_ref[:, kk:kk + 1]
        w_col = ts_ref[:, kk:kk + 1] * prio_lookup(e_col)
        ii = jax.lax.broadcasted_iota(jnp.int32, (n_tok, 1), 0) * k + kk
        parts = []
        for c in range(n_tok // chunk):
            sl = slice(c * chunk, (c + 1) * chunk)
            wi = w_col[sl]
            ei = e_col[sl]
            iic = ii[sl]
            beats = (wf > wi) | ((wf == wi) & (jj < iic))
            cnt = jnp.sum(((ef == ei) & beats).astype(jnp.int32), axis=1,
                          keepdims=True)
            parts.append(cnt)
        rank_cols.append(jnp.concatenate(parts, axis=0))
    rank = jnp.concatenate(rank_cols, axis=1)
    assigned = rank < capacity
    amask_ref[...] = assigned.astype(jnp.int32)
    bpos_ref[...] = jnp.where(assigned, rank, 0).astype(jnp.int32)
    ovf_ref[...] = jnp.logical_not(
        jnp.any(assigned, axis=1, keepdims=True)).astype(jnp.int32)
    e_iota = jax.lax.broadcasted_iota(jnp.int32, (1, n_exp), 1)
    n_e = jnp.zeros((1, n_exp), jnp.int32)
    for kk in range(k):
        n_e += jnp.sum((ti_ref[:, kk:kk + 1] == e_iota).astype(jnp.int32),
                       axis=0, keepdims=True)
    ecnt = jnp.minimum(n_e, capacity)
    cnt_ref[...] = ecnt
    me = ssum_ref[...] * (1.0 / n_tok)
    ce = ecnt.astype(jnp.float32) / float(nk)
    aux_ref[...] = 0.01 * n_exp * jnp.sum(me * ce, keepdims=True).reshape(1, 1)


def kernel(x, positions, cw1, cb1, cw2, cb2, gnorm_w, mw1, mw2, mw3, gproj_w,
           pos_table, temperature, expert_load, expert_utilization,
           expert_priority):
    B, S, H = x.shape
    N = B * S
    E = gproj_w.shape[1]
    K = 2
    CTXF = cw2.shape[1] + pos_table.shape[1]
    GATE_DIM = gproj_w.shape[0]

    x_flat = x.reshape(N, H)
    pos_row = positions.reshape(1, N).astype(jnp.int32)
    xb = x_flat.astype(bf16)
    cw1b, cw2b = cw1.astype(bf16), cw2.astype(bf16)
    mw1b, mw2b, mw3b = mw1.astype(bf16), mw2.astype(bf16), mw3.astype(bf16)
    gprojb = gproj_w.astype(bf16)
    pt_hi = pos_table.astype(bf16)
    pt_lo = (pos_table - pt_hi.astype(f32)).astype(bf16)

    TRA = 512
    feat, nc = pl.pallas_call(
        functools.partial(_ctx_kernel, tr=TRA, n_rows=N),
        grid=(N // TRA,),
        in_specs=[
            pl.BlockSpec((TRA, H), lambda i: (i, 0)),
            pl.BlockSpec(cw1.shape, lambda i: (0, 0)),
            pl.BlockSpec((1, cb1.shape[0]), lambda i: (0, 0)),
            pl.BlockSpec(cw2.shape, lambda i: (0, 0)),
            pl.BlockSpec((1, cb2.shape[0]), lambda i: (0, 0)),
            pl.BlockSpec(pos_table.shape, lambda i: (0, 0)),
            pl.BlockSpec(pos_table.shape, lambda i: (0, 0)),
            pl.BlockSpec((1, N), lambda i: (0, 0)),
        ],
        out_specs=[
            pl.BlockSpec((TRA, CTXF), lambda i: (i, 0)),
            pl.BlockSpec((1, CTXF), lambda i: (0, 0)),
        ],
        out_shape=[
            jax.ShapeDtypeStruct((N, CTXF), f32),
            jax.ShapeDtypeStruct((1, CTXF), f32),
        ],
    )(xb, cw1b, cb1.reshape(1, -1), cw2b, cb2.reshape(1, -1), pt_hi, pt_lo,
      pos_row)

    TRB = 512
    ts, ti, ssum = pl.pallas_call(
        _gate_kernel,
        grid=(N // TRB,),
        in_specs=[
            pl.BlockSpec((TRB, H), lambda i: (i, 0)),
            pl.BlockSpec((TRB, CTXF), lambda i: (i, 0)),
            pl.BlockSpec((1, GATE_DIM), lambda i: (0, 0)),
            pl.BlockSpec(mw1.shape, lambda i: (0, 0)),
            pl.BlockSpec(mw3.shape, lambda i: (0, 0)),
            pl.BlockSpec(mw2.shape, lambda i: (0, 0)),
            pl.BlockSpec(gproj_w.shape, lambda i: (0, 0)),
            pl.BlockSpec((1, E), lambda i: (0, 0)),
            pl.BlockSpec((1, 1), lambda i: (0, 0)),
            pl.BlockSpec((1, E), lambda i: (0, 0)),
        ],
        out_specs=[
            pl.BlockSpec((TRB, K), lambda i: (i, 0)),
            pl.BlockSpec((TRB, K), lambda i: (i, 0)),
            pl.BlockSpec((1, E), lambda i: (0, 0)),
        ],
        out_shape=[
            jax.ShapeDtypeStruct((N, K), f32),
            jax.ShapeDtypeStruct((N, K), jnp.int32),
            jax.ShapeDtypeStruct((1, E), f32),
        ],
    )(x_flat, feat, gnorm_w.reshape(1, -1), mw1b, mw3b, mw2b, gprojb,
      expert_load.reshape(1, -1), temperature.reshape(1, 1),
      expert_priority.reshape(1, -1))

    base_cap = float(math.ceil(N * 1.25 / E))
    amask, bpos, ovf, ecnt, aux = pl.pallas_call(
        functools.partial(_dispatch_kernel, n_tok=N, k=K, n_exp=E,
                          base_cap=base_cap, chunk=512),
        grid=(1,),
        in_specs=[
            pl.BlockSpec((N, K), lambda i: (0, 0)),
            pl.BlockSpec((N, K), lambda i: (0, 0)),
            pl.BlockSpec((1, N * K), lambda i: (0, 0)),
            pl.BlockSpec((1, N * K), lambda i: (0, 0)),
            pl.BlockSpec((1, E), lambda i: (0, 0)),
            pl.BlockSpec((1, E), lambda i: (0, 0)),
            pl.BlockSpec((1, E), lambda i: (0, 0)),
        ],
        out_specs=[
            pl.BlockSpec((N, K), lambda i: (0, 0)),
            pl.BlockSpec((N, K), lambda i: (0, 0)),
            pl.BlockSpec((N, 1), lambda i: (0, 0)),
            pl.BlockSpec((1, E), lambda i: (0, 0)),
            pl.BlockSpec((1, 1), lambda i: (0, 0)),
        ],
        out_shape=[
            jax.ShapeDtypeStruct((N, K), jnp.int32),
            jax.ShapeDtypeStruct((N, K), jnp.int32),
            jax.ShapeDtypeStruct((N, 1), jnp.int32),
            jax.ShapeDtypeStruct((1, E), jnp.int32),
            jax.ShapeDtypeStruct((1, 1), f32),
        ],
    )(ts, ti, ts.reshape(1, N * K), ti.reshape(1, N * K),
      expert_priority.reshape(1, -1),
      expert_utilization.reshape(1, -1), ssum)

    return (ts, ti, amask.astype(bool), bpos, ovf.reshape(N).astype(bool),
            ecnt.reshape(E), aux.reshape(()), nc)


# speedup vs baseline: 1.2199x; 1.0752x over previous
import math
import functools

import jax
import jax.numpy as jnp
from jax.experimental import pallas as pl

bf16 = jnp.bfloat16
f32 = jnp.float32


def _ctx_kernel(x_ref, cw1_ref, cb1_ref, cw2_ref, cb2_ref, pth_ref, ptl_ref,
                pos_ref, feat_ref, nc_ref, *, tr, n_rows):
    i = pl.program_id(0)
    h = jnp.dot(x_ref[...], cw1_ref[...],
                preferred_element_type=f32) + cb1_ref[...]
    h = 0.5 * h * (1.0 + jax.lax.erf(h * (2.0 ** -0.5)))
    ctx = jnp.dot(h.astype(bf16), cw2_ref[...],
                  preferred_element_type=f32) + cb2_ref[...]
    pos = pos_ref[0, pl.ds(i * tr, tr)].reshape(tr, 1)
    vocab = pth_ref.shape[0]
    oh = (pos == jax.lax.broadcasted_iota(jnp.int32, (tr, vocab), 1))
    ohb = oh.astype(bf16)
    pe = (jnp.dot(ohb, pth_ref[...], preferred_element_type=f32) +
          jnp.dot(ohb, ptl_ref[...], preferred_element_type=f32))
    feat = jnp.concatenate([ctx, pe], axis=1)
    feat_ref[...] = feat

    @pl.when(i == 0)
    def _():
        nc_ref[...] = jnp.zeros_like(nc_ref)

    nc_ref[...] += jnp.sum(feat, axis=0, keepdims=True) * (1.0 / n_rows)


def _gate_kernel(x_ref, feat_ref, gnorm_ref, mw1_ref, mw3_ref, mw2_ref,
                 gproj_ref, load_ref, temp_ref, prio_ref,
                 ts_ref, ti_ref, ssum_ref):
    i = pl.program_id(0)
    gcat = jnp.concatenate([x_ref[...], feat_ref[...]], axis=1)
    ms = jnp.mean(gcat * gcat, axis=1, keepdims=True)
    gi = gcat * jax.lax.rsqrt(ms + 1e-6) * gnorm_ref[...]
    gib = gi.astype(bf16)
    h1 = jnp.dot(gib, mw1_ref[...], preferred_element_type=f32)
    h3 = jnp.dot(gib, mw3_ref[...], preferred_element_type=f32)
    hp = (jax.nn.silu(h1) * h3).astype(bf16)
    m = jnp.dot(hp, mw2_ref[...], preferred_element_type=f32)
    gh = m + gi
    logits = jnp.dot(gh.astype(bf16), gproj_ref[...],
                     preferred_element_type=f32) + jnp.log(prio_ref[...])
    load = load_ref[...]
    lmean = jnp.mean(load)
    lstd = jnp.sqrt(jnp.mean((load - lmean) ** 2))
    imb = lstd / (lmean + 1e-6)
    temp = jnp.maximum(temp_ref[0, 0] * (1.0 + imb), 0.3)
    lt = logits / temp
    lmax = jnp.max(lt, axis=1, keepdims=True)
    ex = jnp.exp(lt - lmax)
    sc = ex / jnp.sum(ex, axis=1, keepdims=True)
    e_iota = jax.lax.broadcasted_iota(jnp.int32, sc.shape, 1)
    i1 = jnp.argmax(sc, axis=1).astype(jnp.int32)
    s1 = jnp.max(sc, axis=1)
    masked = jnp.where(e_iota == i1[:, None], -1.0, sc)
    i2 = jnp.argmax(masked, axis=1).astype(jnp.int32)
    s2 = jnp.max(masked, axis=1)
    ts_ref[...] = jnp.concatenate([s1[:, None], s2[:, None]], axis=1)
    ti_ref[...] = jnp.concatenate([i1[:, None], i2[:, None]], axis=1)

    @pl.when(i == 0)
    def _():
        ssum_ref[...] = jnp.zeros_like(ssum_ref)

    ssum_ref[...] += jnp.sum(sc, axis=0, keepdims=True)


def _dispatch_kernel(ts_ref, ti_ref, tsf_ref, tif_ref, prio_ref, util_ref,
                     ssum_ref, amask_ref, bpos_ref, ovf_ref, cnt_ref, aux_ref,
                     *, n_tok, k, n_exp, base_cap, chunk):
    nk = n_tok * k
    prio_c = jnp.clip(prio_ref[...], 1.0, 2.0)

    def prio_lookup(ev):
        out = jnp.zeros(ev.shape, jnp.float32)
        for v in range(n_exp):
            out = jnp.where(ev == v, prio_c[0, v], out)
        return out

    ef = tif_ref[...]
    wf = tsf_ref[...] * prio_lookup(ef)
    jj = jax.lax.broadcasted_iota(jnp.int32, (1, nk), 1)
    u = jnp.clip(util_ref[...], 0.1, 0.9)
    uf = jnp.mean(1.0 / (u + 0.1))
    capacity = jnp.clip((base_cap * uf).astype(jnp.int32), 4, 2048)
    rank_cols = []
    for kk in range(k):
        e_col = ti_ref[:, kk:kk + 1]
        w_col = ts_ref[:, kk:kk + 1] * prio_lookup(e_col)
        ii = jax.lax.broadcasted_iota(jnp.int32, (n_tok, 1), 0) * k + kk
        parts = []
        for c in range(n_tok // chunk):
            sl = slice(c * chunk, (c + 1) * chunk)
            wi = w_col[sl]
            ei = e_col[sl]
            iic = ii[sl]
            cj0, cj1 = c * chunk * k, (c + 1) * chunk * k
            cnt = jnp.zeros((chunk, 1), jnp.int32)
            if cj0 > 0:
                lo = slice(0, cj0)
                beats = (wf[:, lo] >= wi) & (ef[:, lo] == ei)
                cnt += jnp.sum(beats.astype(jnp.int32), axis=1, keepdims=True)
            if cj1 < nk:
                hi = slice(cj1, nk)
                beats = (wf[:, hi] > wi) & (ef[:, hi] == ei)
                cnt += jnp.sum(beats.astype(jnp.int32), axis=1, keepdims=True)
            dg = slice(cj0, cj1)
            wfd, efd, jjd = wf[:, dg], ef[:, dg], jj[:, dg]
            beats = ((wfd > wi) | ((wfd == wi) & (jjd < iic))) & (efd == ei)
            cnt += jnp.sum(beats.astype(jnp.int32), axis=1, keepdims=True)
            parts.append(cnt)
        rank_cols.append(jnp.concatenate(parts, axis=0))
    rank = jnp.concatenate(rank_cols, axis=1)
    assigned = rank < capacity
    amask_ref[...] = assigned.astype(jnp.int32)
    bpos_ref[...] = jnp.where(assigned, rank, 0).astype(jnp.int32)
    ovf_ref[...] = jnp.logical_not(
        jnp.any(assigned, axis=1, keepdims=True)).astype(jnp.int32)
    e_iota = jax.lax.broadcasted_iota(jnp.int32, (1, n_exp), 1)
    n_e = jnp.zeros((1, n_exp), jnp.int32)
    for kk in range(k):
        n_e += jnp.sum((ti_ref[:, kk:kk + 1] == e_iota).astype(jnp.int32),
                       axis=0, keepdims=True)
    ecnt = jnp.minimum(n_e, capacity)
    cnt_ref[...] = ecnt
    me = ssum_ref[...] * (1.0 / n_tok)
    ce = ecnt.astype(jnp.float32) / float(nk)
    aux_ref[...] = 0.01 * n_exp * jnp.sum(me * ce, keepdims=True).reshape(1, 1)


def kernel(x, positions, cw1, cb1, cw2, cb2, gnorm_w, mw1, mw2, mw3, gproj_w,
           pos_table, temperature, expert_load, expert_utilization,
           expert_priority):
    B, S, H = x.shape
    N = B * S
    E = gproj_w.shape[1]
    K = 2
    CTXF = cw2.shape[1] + pos_table.shape[1]
    GATE_DIM = gproj_w.shape[0]

    x_flat = x.reshape(N, H)
    pos_row = positions.reshape(1, N).astype(jnp.int32)
    xb = x_flat.astype(bf16)
    cw1b, cw2b = cw1.astype(bf16), cw2.astype(bf16)
    mw1b, mw2b, mw3b = mw1.astype(bf16), mw2.astype(bf16), mw3.astype(bf16)
    gprojb = gproj_w.astype(bf16)
    pt_hi = pos_table.astype(bf16)
    pt_lo = (pos_table - pt_hi.astype(f32)).astype(bf16)

    TRA = 512
    feat, nc = pl.pallas_call(
        functools.partial(_ctx_kernel, tr=TRA, n_rows=N),
        grid=(N // TRA,),
        in_specs=[
            pl.BlockSpec((TRA, H), lambda i: (i, 0)),
            pl.BlockSpec(cw1.shape, lambda i: (0, 0)),
            pl.BlockSpec((1, cb1.shape[0]), lambda i: (0, 0)),
            pl.BlockSpec(cw2.shape, lambda i: (0, 0)),
            pl.BlockSpec((1, cb2.shape[0]), lambda i: (0, 0)),
            pl.BlockSpec(pos_table.shape, lambda i: (0, 0)),
            pl.BlockSpec(pos_table.shape, lambda i: (0, 0)),
            pl.BlockSpec((1, N), lambda i: (0, 0)),
        ],
        out_specs=[
            pl.BlockSpec((TRA, CTXF), lambda i: (i, 0)),
            pl.BlockSpec((1, CTXF), lambda i: (0, 0)),
        ],
        out_shape=[
            jax.ShapeDtypeStruct((N, CTXF), f32),
            jax.ShapeDtypeStruct((1, CTXF), f32),
        ],
    )(xb, cw1b, cb1.reshape(1, -1), cw2b, cb2.reshape(1, -1), pt_hi, pt_lo,
      pos_row)

    TRB = 512
    ts, ti, ssum = pl.pallas_call(
        _gate_kernel,
        grid=(N // TRB,),
        in_specs=[
            pl.BlockSpec((TRB, H), lambda i: (i, 0)),
            pl.BlockSpec((TRB, CTXF), lambda i: (i, 0)),
            pl.BlockSpec((1, GATE_DIM), lambda i: (0, 0)),
            pl.BlockSpec(mw1.shape, lambda i: (0, 0)),
            pl.BlockSpec(mw3.shape, lambda i: (0, 0)),
            pl.BlockSpec(mw2.shape, lambda i: (0, 0)),
            pl.BlockSpec(gproj_w.shape, lambda i: (0, 0)),
            pl.BlockSpec((1, E), lambda i: (0, 0)),
            pl.BlockSpec((1, 1), lambda i: (0, 0)),
            pl.BlockSpec((1, E), lambda i: (0, 0)),
        ],
        out_specs=[
            pl.BlockSpec((TRB, K), lambda i: (i, 0)),
            pl.BlockSpec((TRB, K), lambda i: (i, 0)),
            pl.BlockSpec((1, E), lambda i: (0, 0)),
        ],
        out_shape=[
            jax.ShapeDtypeStruct((N, K), f32),
            jax.ShapeDtypeStruct((N, K), jnp.int32),
            jax.ShapeDtypeStruct((1, E), f32),
        ],
    )(x_flat, feat, gnorm_w.reshape(1, -1), mw1b, mw3b, mw2b, gprojb,
      expert_load.reshape(1, -1), temperature.reshape(1, 1),
      expert_priority.reshape(1, -1))

    base_cap = float(math.ceil(N * 1.25 / E))
    amask, bpos, ovf, ecnt, aux = pl.pallas_call(
        functools.partial(_dispatch_kernel, n_tok=N, k=K, n_exp=E,
                          base_cap=base_cap, chunk=512),
        grid=(1,),
        in_specs=[
            pl.BlockSpec((N, K), lambda i: (0, 0)),
            pl.BlockSpec((N, K), lambda i: (0, 0)),
            pl.BlockSpec((1, N * K), lambda i: (0, 0)),
            pl.BlockSpec((1, N * K), lambda i: (0, 0)),
            pl.BlockSpec((1, E), lambda i: (0, 0)),
            pl.BlockSpec((1, E), lambda i: (0, 0)),
            pl.BlockSpec((1, E), lambda i: (0, 0)),
        ],
        out_specs=[
            pl.BlockSpec((N, K), lambda i: (0, 0)),
            pl.BlockSpec((N, K), lambda i: (0, 0)),
            pl.BlockSpec((N, 1), lambda i: (0, 0)),
            pl.BlockSpec((1, E), lambda i: (0, 0)),
            pl.BlockSpec((1, 1), lambda i: (0, 0)),
        ],
        out_shape=[
            jax.ShapeDtypeStruct((N, K), jnp.int32),
            jax.ShapeDtypeStruct((N, K), jnp.int32),
            jax.ShapeDtypeStruct((N, 1), jnp.int32),
            jax.ShapeDtypeStruct((1, E), jnp.int32),
            jax.ShapeDtypeStruct((1, 1), f32),
        ],
    )(ts, ti, ts.reshape(1, N * K), ti.reshape(1, N * K),
      expert_priority.reshape(1, -1),
      expert_utilization.reshape(1, -1), ssum)

    return (ts, ti, amask.astype(bool), bpos, ovf.reshape(N).astype(bool),
            ecnt.reshape(E), aux.reshape(()), nc)


# speedup vs baseline: 1.4321x; 1.1740x over previous
import math
import functools

import jax
import jax.numpy as jnp
from jax.experimental import pallas as pl

bf16 = jnp.bfloat16
f32 = jnp.float32


def _ctx_kernel(x_ref, cw1_ref, cb1_ref, cw2_ref, cb2_ref, pt_ref, pos_ref,
                mw1_ref, mw3_ref, mw2_ref,
                feat_ref, nc_ref, mw1b_ref, mw3b_ref, mw2b_ref, cw1b_ref,
                *, tr, n_rows):
    i = pl.program_id(0)
    mw1b_ref[...] = mw1_ref[...].astype(bf16)
    mw3b_ref[...] = mw3_ref[...].astype(bf16)
    mw2b_ref[...] = mw2_ref[...].astype(bf16)

    @pl.when(i == 0)
    def _():
        cw1b_ref[...] = cw1_ref[...].astype(bf16)

    h = jnp.dot(x_ref[...].astype(bf16), cw1b_ref[...],
                preferred_element_type=f32) + cb1_ref[...]
    h = 0.5 * h * (1.0 + jax.lax.erf(h * (2.0 ** -0.5)))
    ctx = jnp.dot(h.astype(bf16), cw2_ref[...].astype(bf16),
                  preferred_element_type=f32) + cb2_ref[...]
    pt = pt_ref[...]
    pt_hi = pt.astype(bf16)
    pt_lo = (pt - pt_hi.astype(f32)).astype(bf16)
    pos = pos_ref[0, pl.ds(i * tr, tr)].reshape(tr, 1)
    vocab = pt_ref.shape[0]
    oh = (pos == jax.lax.broadcasted_iota(jnp.int32, (tr, vocab), 1))
    ohb = oh.astype(bf16)
    pe = (jnp.dot(ohb, pt_hi, preferred_element_type=f32) +
          jnp.dot(ohb, pt_lo, preferred_element_type=f32))
    feat = jnp.concatenate([ctx, pe], axis=1)
    feat_ref[...] = feat

    @pl.when(i == 0)
    def _():
        nc_ref[...] = jnp.zeros_like(nc_ref)

    nc_ref[...] += jnp.sum(feat, axis=0, keepdims=True) * (1.0 / n_rows)


def _gate_kernel(x_ref, feat_ref, gnorm_ref, mw1_ref, mw3_ref, mw2_ref,
                 gproj_ref, load_ref, temp_ref, prio_ref,
                 ts_ref, ti_ref, ssum_ref):
    i = pl.program_id(0)
    gcat = jnp.concatenate([x_ref[...], feat_ref[...]], axis=1)
    ms = jnp.mean(gcat * gcat, axis=1, keepdims=True)
    gi = gcat * jax.lax.rsqrt(ms + 1e-6) * gnorm_ref[...]
    gib = gi.astype(bf16)
    h1 = jnp.dot(gib, mw1_ref[...], preferred_element_type=f32)
    h3 = jnp.dot(gib, mw3_ref[...], preferred_element_type=f32)
    hp = (jax.nn.silu(h1) * h3).astype(bf16)
    m = jnp.dot(hp, mw2_ref[...], preferred_element_type=f32)
    gh = m + gi
    logits = jnp.dot(gh.astype(bf16), gproj_ref[...].astype(bf16),
                     preferred_element_type=f32) + jnp.log(prio_ref[...])
    load = load_ref[...]
    lmean = jnp.mean(load)
    lstd = jnp.sqrt(jnp.mean((load - lmean) ** 2))
    imb = lstd / (lmean + 1e-6)
    temp = jnp.maximum(temp_ref[0, 0] * (1.0 + imb), 0.3)
    lt = logits / temp
    lmax = jnp.max(lt, axis=1, keepdims=True)
    ex = jnp.exp(lt - lmax)
    sc = ex / jnp.sum(ex, axis=1, keepdims=True)
    e_iota = jax.lax.broadcasted_iota(jnp.int32, sc.shape, 1)
    i1 = jnp.argmax(sc, axis=1).astype(jnp.int32)
    s1 = jnp.max(sc, axis=1)
    masked = jnp.where(e_iota == i1[:, None], -1.0, sc)
    i2 = jnp.argmax(masked, axis=1).astype(jnp.int32)
    s2 = jnp.max(masked, axis=1)
    ts_ref[...] = jnp.concatenate([s1[:, None], s2[:, None]], axis=1)
    ti_ref[...] = jnp.concatenate([i1[:, None], i2[:, None]], axis=1)

    @pl.when(i == 0)
    def _():
        ssum_ref[...] = jnp.zeros_like(ssum_ref)

    ssum_ref[...] += jnp.sum(sc, axis=0, keepdims=True)


def _dispatch_kernel(ts_ref, ti_ref, tsf_ref, tif_ref, prio_ref, util_ref,
                     ssum_ref, amask_ref, bpos_ref, ovf_ref, cnt_ref, aux_ref,
                     *, n_tok, k, n_exp, base_cap, chunk):
    nk = n_tok * k
    prio_c = jnp.clip(prio_ref[...], 1.0, 2.0)

    def prio_lookup(ev):
        out = jnp.zeros(ev.shape, jnp.float32)
        for v in range(n_exp):
            out = jnp.where(ev == v, prio_c[0, v], out)
        return out

    ef = tif_ref[...]
    wf = tsf_ref[...] * prio_lookup(ef)
    jj = jax.lax.broadcasted_iota(jnp.int32, (1, nk), 1)
    u = jnp.clip(util_ref[...], 0.1, 0.9)
    uf = jnp.mean(1.0 / (u + 0.1))
    capacity = jnp.clip((base_cap * uf).astype(jnp.int32), 4, 2048)
    rank_cols = []
    for kk in range(k):
        e_col = ti_ref[:, kk:kk + 1]
        w_col = ts_ref[:, kk:kk + 1] * prio_lookup(e_col)
        ii = jax.lax.broadcasted_iota(jnp.int32, (n_tok, 1), 0) * k + kk
        parts = []
        for c in range(n_tok // chunk):
            sl = slice(c * chunk, (c + 1) * chunk)
            wi = w_col[sl]
            ei = e_col[sl]
            iic = ii[sl]
            cj0, cj1 = c * chunk * k, (c + 1) * chunk * k
            cnt = jnp.zeros((chunk, 1), jnp.int32)
            if cj0 > 0:
                lo = slice(0, cj0)
                beats = (wf[:, lo] >= wi) & (ef[:, lo] == ei)
                cnt += jnp.sum(beats.astype(jnp.int32), axis=1, keepdims=True)
            if cj1 < nk:
                hi = slice(cj1, nk)
                beats = (wf[:, hi] > wi) & (ef[:, hi] == ei)
                cnt += jnp.sum(beats.astype(jnp.int32), axis=1, keepdims=True)
            dg = slice(cj0, cj1)
            wfd, efd, jjd = wf[:, dg], ef[:, dg], jj[:, dg]
            beats = ((wfd > wi) | ((wfd == wi) & (jjd < iic))) & (efd == ei)
            cnt += jnp.sum(beats.astype(jnp.int32), axis=1, keepdims=True)
            parts.append(cnt)
        rank_cols.append(jnp.concatenate(parts, axis=0))
    rank = jnp.concatenate(rank_cols, axis=1)
    assigned = rank < capacity
    amask_ref[...] = assigned.astype(jnp.int32)
    bpos_ref[...] = jnp.where(assigned, rank, 0).astype(jnp.int32)
    ovf_ref[...] = jnp.logical_not(
        jnp.any(assigned, axis=1, keepdims=True)).astype(jnp.int32)
    e_iota = jax.lax.broadcasted_iota(jnp.int32, (1, n_exp), 1)
    n_e = jnp.zeros((1, n_exp), jnp.int32)
    for kk in range(k):
        n_e += jnp.sum((ti_ref[:, kk:kk + 1] == e_iota).astype(jnp.int32),
                       axis=0, keepdims=True)
    ecnt = jnp.minimum(n_e, capacity)
    cnt_ref[...] = ecnt
    me = ssum_ref[...] * (1.0 / n_tok)
    ce = ecnt.astype(jnp.float32) / float(nk)
    aux_ref[...] = 0.01 * n_exp * jnp.sum(me * ce, keepdims=True).reshape(1, 1)


def kernel(x, positions, cw1, cb1, cw2, cb2, gnorm_w, mw1, mw2, mw3, gproj_w,
           pos_table, temperature, expert_load, expert_utilization,
           expert_priority):
    B, S, H = x.shape
    N = B * S
    E = gproj_w.shape[1]
    K = 2
    CTXF = cw2.shape[1] + pos_table.shape[1]
    GATE_DIM = gproj_w.shape[0]

    x_flat = x.reshape(N, H)
    pos_row = positions.reshape(1, N).astype(jnp.int32)
    HID = mw1.shape[1]

    TRA = 256
    nblk = N // TRA
    hid_c = HID // nblk
    m2r_c = mw2.shape[0] // nblk
    feat, nc, mw1b, mw3b, mw2b, _cw1b = pl.pallas_call(
        functools.partial(_ctx_kernel, tr=TRA, n_rows=N),
        grid=(nblk,),
        in_specs=[
            pl.BlockSpec((TRA, H), lambda i: (i, 0)),
            pl.BlockSpec(cw1.shape, lambda i: (0, 0)),
            pl.BlockSpec((1, cb1.shape[0]), lambda i: (0, 0)),
            pl.BlockSpec(cw2.shape, lambda i: (0, 0)),
            pl.BlockSpec((1, cb2.shape[0]), lambda i: (0, 0)),
            pl.BlockSpec(pos_table.shape, lambda i: (0, 0)),
            pl.BlockSpec((1, N), lambda i: (0, 0)),
            pl.BlockSpec((GATE_DIM, hid_c), lambda i: (0, i)),
            pl.BlockSpec((GATE_DIM, hid_c), lambda i: (0, i)),
            pl.BlockSpec((m2r_c, GATE_DIM), lambda i: (i, 0)),
        ],
        out_specs=[
            pl.BlockSpec((TRA, CTXF), lambda i: (i, 0)),
            pl.BlockSpec((1, CTXF), lambda i: (0, 0)),
            pl.BlockSpec((GATE_DIM, hid_c), lambda i: (0, i)),
            pl.BlockSpec((GATE_DIM, hid_c), lambda i: (0, i)),
            pl.BlockSpec((m2r_c, GATE_DIM), lambda i: (i, 0)),
            pl.BlockSpec(cw1.shape, lambda i: (0, 0)),
        ],
        out_shape=[
            jax.ShapeDtypeStruct((N, CTXF), f32),
            jax.ShapeDtypeStruct((1, CTXF), f32),
            jax.ShapeDtypeStruct(mw1.shape, bf16),
            jax.ShapeDtypeStruct(mw3.shape, bf16),
            jax.ShapeDtypeStruct(mw2.shape, bf16),
            jax.ShapeDtypeStruct(cw1.shape, bf16),
        ],
    )(x_flat, cw1, cb1.reshape(1, -1), cw2, cb2.reshape(1, -1), pos_table,
      pos_row, mw1, mw3, mw2)

    TRB = 512
    ts, ti, ssum = pl.pallas_call(
        _gate_kernel,
        grid=(N // TRB,),
        in_specs=[
            pl.BlockSpec((TRB, H), lambda i: (i, 0)),
            pl.BlockSpec((TRB, CTXF), lambda i: (i, 0)),
            pl.BlockSpec((1, GATE_DIM), lambda i: (0, 0)),
            pl.BlockSpec(mw1.shape, lambda i: (0, 0)),
            pl.BlockSpec(mw3.shape, lambda i: (0, 0)),
            pl.BlockSpec(mw2.shape, lambda i: (0, 0)),
            pl.BlockSpec(gproj_w.shape, lambda i: (0, 0)),
            pl.BlockSpec((1, E), lambda i: (0, 0)),
            pl.BlockSpec((1, 1), lambda i: (0, 0)),
            pl.BlockSpec((1, E), lambda i: (0, 0)),
        ],
        out_specs=[
            pl.BlockSpec((TRB, K), lambda i: (i, 0)),
            pl.BlockSpec((TRB, K), lambda i: (i, 0)),
            pl.BlockSpec((1, E), lambda i: (0, 0)),
        ],
        out_shape=[
            jax.ShapeDtypeStruct((N, K), f32),
            jax.ShapeDtypeStruct((N, K), jnp.int32),
            jax.ShapeDtypeStruct((1, E), f32),
        ],
    )(x_flat, feat, gnorm_w.reshape(1, -1), mw1b, mw3b, mw2b, gproj_w,
      expert_load.reshape(1, -1), temperature.reshape(1, 1),
      expert_priority.reshape(1, -1))

    base_cap = float(math.ceil(N * 1.25 / E))
    amask, bpos, ovf, ecnt, aux = pl.pallas_call(
        functools.partial(_dispatch_kernel, n_tok=N, k=K, n_exp=E,
                          base_cap=base_cap, chunk=512),
        grid=(1,),
        in_specs=[
            pl.BlockSpec((N, K), lambda i: (0, 0)),
            pl.BlockSpec((N, K), lambda i: (0, 0)),
            pl.BlockSpec((1, N * K), lambda i: (0, 0)),
            pl.BlockSpec((1, N * K), lambda i: (0, 0)),
            pl.BlockSpec((1, E), lambda i: (0, 0)),
            pl.BlockSpec((1, E), lambda i: (0, 0)),
            pl.BlockSpec((1, E), lambda i: (0, 0)),
        ],
        out_specs=[
            pl.BlockSpec((N, K), lambda i: (0, 0)),
            pl.BlockSpec((N, K), lambda i: (0, 0)),
            pl.BlockSpec((N, 1), lambda i: (0, 0)),
            pl.BlockSpec((1, E), lambda i: (0, 0)),
            pl.BlockSpec((1, 1), lambda i: (0, 0)),
        ],
        out_shape=[
            jax.ShapeDtypeStruct((N, K), jnp.int32),
            jax.ShapeDtypeStruct((N, K), jnp.int32),
            jax.ShapeDtypeStruct((N, 1), jnp.int32),
            jax.ShapeDtypeStruct((1, E), jnp.int32),
            jax.ShapeDtypeStruct((1, 1), f32),
        ],
    )(ts, ti, ts.reshape(1, N * K), ti.reshape(1, N * K),
      expert_priority.reshape(1, -1),
      expert_utilization.reshape(1, -1), ssum)

    return (ts, ti, amask.astype(bool), bpos, ovf.reshape(N).astype(bool),
            ecnt.reshape(E), aux.reshape(()), nc)
